# baseline (speedup 1.0000x reference)
import jax
import jax.numpy as jnp
from jax.experimental import pallas as pl


def _mm_body(mask_ref, a_ref, w_ref, b_ref, o_ref):
    acc = jax.lax.dot_general(
        a_ref[...], w_ref[...], (((1,), (1,)), ((), ())),
        preferred_element_type=jnp.float32)
    o_ref[...] = (acc + b_ref[...]) * mask_ref[...]


def kernel(indicators, data, W, b):
    N, d_in = data.shape
    d_out = W.shape[0]
    BM = 512
    maskf = (indicators != 0).astype(jnp.float32).reshape(N, 1)
    out = pl.pallas_call(
        _mm_body,
        grid=(N // BM,),
        in_specs=[
            pl.BlockSpec((BM, 1), lambda i: (i, 0)),
            pl.BlockSpec((BM, d_in), lambda i: (i, 0)),
            pl.BlockSpec((d_out, d_in), lambda i: (0, 0)),
            pl.BlockSpec((1, d_out), lambda i: (0, 0)),
        ],
        out_specs=pl.BlockSpec((BM, d_out), lambda i: (i, 0)),
        out_shape=jax.ShapeDtypeStruct((N, d_out), jnp.float32),
    )(maskf, data, W, b.reshape(1, d_out))
    return out
